# baseline (speedup 1.0000x reference)
import functools

import numpy as np

import jax
import jax.numpy as jnp
from jax import lax
from jax.experimental import pallas as pl
from jax.experimental.pallas import tpu as pltpu
from jax.experimental.pallas import tpu_sc as plsc

B, L, T, D = 1024, 50, 20, 64
N = B * L
NC, NS = 2, 16
NW = NC * NS
POS_PER_W = N // NW
CHUNK = 32
N_CHUNKS = POS_PER_W // CHUNK
IDX_ROWS = (CHUNK * T) // 128
UPM_V = 102
DP = D // 2


def _bf16_halves(w):
    ev = plsc.bitcast(jnp.left_shift(w, 16), jnp.float32)
    od = plsc.bitcast(jnp.bitwise_and(w, jnp.int32(-65536)), jnp.float32)
    return ev, od


def _sc_body(wmeta, wbrand, wcat, wupm, wdsub, midx, bidx, cidx,
             uidx, pidx, price, z1_out, z2_out,
             midx_v, bidx_all, cidx_all, uidx_all, pidx_all, price_all,
             wupm_v, wdsub_v, gbuf, bbuf, cbuf, z1buf, z2buf,
             sem_g, sem_i, sem_o):
    wid = lax.axis_index("s") * NC + lax.axis_index("c")
    base = wid * POS_PER_W
    chunk_base = wid * N_CHUNKS

    def fire_chunk(g_rel, p):
        for j in range(IDX_ROWS):
            pltpu.async_copy(
                wmeta.at[midx_v.at[p, j]],
                gbuf.at[p].at[pl.ds(j * 128, 128)], sem_g)
        loc = pl.ds(g_rel * CHUNK, CHUNK)
        pltpu.async_copy(wbrand.at[bidx_all.at[loc]], bbuf.at[p], sem_g)
        pltpu.async_copy(wcat.at[cidx_all.at[loc]], cbuf.at[p], sem_g)

    def wait_chunk(p):
        for j in range(IDX_ROWS):
            pltpu.make_async_copy(
                wmeta.at[midx_v.at[p, j]],
                gbuf.at[p].at[pl.ds(j * 128, 128)], sem_g).wait()
        pltpu.make_async_copy(wbrand.at[bidx_all.at[pl.ds(0, CHUNK)]],
                              bbuf.at[p], sem_g).wait()
        pltpu.make_async_copy(wcat.at[cidx_all.at[pl.ds(0, CHUNK)]],
                              cbuf.at[p], sem_g).wait()

    pltpu.sync_copy(bidx.at[pl.ds(base, POS_PER_W)], bidx_all)
    pltpu.sync_copy(cidx.at[pl.ds(base, POS_PER_W)], cidx_all)
    pltpu.sync_copy(uidx.at[pl.ds(base, POS_PER_W)],
                    uidx_all.at[pl.ds(0, POS_PER_W)])
    pltpu.sync_copy(pidx.at[pl.ds(base, POS_PER_W)],
                    pidx_all.at[pl.ds(0, POS_PER_W)])
    pltpu.sync_copy(price.at[pl.ds(base, POS_PER_W)],
                    price_all.at[pl.ds(0, POS_PER_W)])
    pltpu.sync_copy(wupm, wupm_v)
    pltpu.sync_copy(wdsub, wdsub_v)

    zeros16 = jnp.zeros((16,), jnp.float32)

    def zero_body(i, carry):
        for pp in range(2):
            for u in range(D // 16):
                z2buf[pp, i, pl.ds(D + u * 16, 16)] = zeros16
        return carry

    lax.fori_loop(0, CHUNK, zero_body, 0)

    pltpu.sync_copy(midx.at[chunk_base], midx_v.at[0])
    fire_chunk(0, 0)
    pltpu.async_copy(midx.at[chunk_base + 1], midx_v.at[1], sem_i)

    r_vecs = [wdsub_v[0, pl.ds(u * 16, 16)] for u in range(D // 16)]

    def outer_body(gg, carry):
        for b in range(2):
            g = gg * 2 + b
            p, np_ = b, 1 - b
            wait_chunk(p)

            @pl.when(g < N_CHUNKS - 1)
            def _():
                pltpu.make_async_copy(
                    midx.at[chunk_base + 1], midx_v.at[np_], sem_i).wait()
                fire_chunk(g + 1, np_)

            @pl.when(g < N_CHUNKS - 2)
            def _():
                pltpu.async_copy(
                    midx.at[chunk_base + g + 2], midx_v.at[p], sem_i)

            @pl.when(g >= 2)
            def _():
                pltpu.make_async_copy(
                    z1buf.at[p], z1_out.at[pl.ds(base, CHUNK)], sem_o).wait()
                pltpu.make_async_copy(
                    z2buf.at[p], z2_out.at[pl.ds(base, CHUNK)], sem_o).wait()

            def pos_body(i, carry2):
                base_row = i * T
                loc = g * CHUNK + i
                pu = uidx_all[pl.ds(loc, 16)][0]
                pg = pidx_all[pl.ds(loc, 16)][0]
                pr = price_all[pl.ds(loc, 16)][0]
                for v in range(DP // 16):
                    col = pl.ds(v * 16, 16)
                    ae, ao = _bf16_halves(gbuf[p, base_row, col])
                    for t in range(1, T):
                        he, ho = _bf16_halves(gbuf[p, base_row + t, col])
                        ae = ae + he
                        ao = ao + ho
                    be, bo = _bf16_halves(bbuf[p, i, col])
                    ce, co = _bf16_halves(cbuf[p, i, col])
                    z1buf[p, i, pl.ds(v * 32, 16)] = (
                        ae * (1.0 / T) + be + ce)
                    z1buf[p, i, pl.ds(v * 32 + 16, 16)] = (
                        ao * (1.0 / T) + bo + co)
                for u in range(D // 16):
                    col = pl.ds(u * 16, 16)
                    z1buf[p, i, pl.ds(D + u * 16, 16)] = wupm_v[pu, col]
                    z2buf[p, i, col] = wdsub_v[1 + pg, col] + pr * r_vecs[u]
                return carry2

            lax.fori_loop(0, CHUNK, pos_body, 0)
            pltpu.async_copy(
                z1buf.at[p], z1_out.at[pl.ds(base + g * CHUNK, CHUNK)], sem_o)
            pltpu.async_copy(
                z2buf.at[p], z2_out.at[pl.ds(base + g * CHUNK, CHUNK)], sem_o)
        return carry

    lax.fori_loop(0, N_CHUNKS // 2, outer_body, 0)
    for _ in range(2):
        pltpu.make_async_copy(
            z1buf.at[0], z1_out.at[pl.ds(base, CHUNK)], sem_o).wait()
        pltpu.make_async_copy(
            z2buf.at[0], z2_out.at[pl.ds(base, CHUNK)], sem_o).wait()


@jax.jit
def _sc_gather_sum(wmeta, wbrand, wcat, wupm, wdsub, midx, bidx, cidx,
                   uidx, pidx, price):
    mesh = plsc.VectorSubcoreMesh(core_axis_name="c", subcore_axis_name="s")
    return pl.kernel(
        _sc_body,
        out_type=(jax.ShapeDtypeStruct((N, 2 * D), jnp.float32),
                  jax.ShapeDtypeStruct((N, 2 * D), jnp.float32)),
        mesh=mesh,
        compiler_params=pltpu.CompilerParams(use_tc_tiling_on_sc=False,
                                             needs_layout_passes=False),
        scratch_types=[
            pltpu.VMEM((2, IDX_ROWS, 128), jnp.int32),
            pltpu.VMEM((POS_PER_W,), jnp.int32),
            pltpu.VMEM((POS_PER_W,), jnp.int32),
            pltpu.VMEM((POS_PER_W + 16,), jnp.int32),
            pltpu.VMEM((POS_PER_W + 16,), jnp.int32),
            pltpu.VMEM((POS_PER_W + 16,), jnp.float32),
            pltpu.VMEM((UPM_V, D), jnp.float32),
            pltpu.VMEM((33, D), jnp.float32),
            pltpu.VMEM((2, CHUNK * T, DP), jnp.int32),
            pltpu.VMEM((2, CHUNK, DP), jnp.int32),
            pltpu.VMEM((2, CHUNK, DP), jnp.int32),
            pltpu.VMEM((2, CHUNK, 2 * D), jnp.float32),
            pltpu.VMEM((2, CHUNK, 2 * D), jnp.float32),
            pltpu.SemaphoreType.DMA,
            pltpu.SemaphoreType.DMA,
            pltpu.SemaphoreType.DMA,
        ],
    )(wmeta, wbrand, wcat, wupm, wdsub, midx, bidx, cidx, uidx, pidx, price)


RB = 64
TC_R = RB * L


def _tc_body(z1_ref, z2_ref, w2_ref, b_ref, o_ref):
    acc = jnp.dot(z1_ref[:], w2_ref[:], preferred_element_type=jnp.float32)
    acc += z2_ref[:][:, :D]
    acc += b_ref[:]
    o_ref[:] = acc.reshape(RB, L, D)


@jax.jit
def _tc_dense(z1, z2, w2, b):
    return pl.pallas_call(
        _tc_body,
        grid=(B // RB,),
        in_specs=[
            pl.BlockSpec((TC_R, 2 * D), lambda i: (i, 0)),
            pl.BlockSpec((TC_R, 2 * D), lambda i: (i, 0)),
            pl.BlockSpec((2 * D, D), lambda i: (0, 0)),
            pl.BlockSpec((1, D), lambda i: (0, 0)),
        ],
        out_specs=pl.BlockSpec((RB, L, D), lambda i: (i, 0, 0)),
        out_shape=jax.ShapeDtypeStruct((B, L, D), jnp.float32),
    )(z1, z2, w2, b)


def _pack_bf16(w):
    xi = lax.bitcast_convert_type(w.astype(jnp.bfloat16),
                                  jnp.uint16).astype(jnp.uint32)
    packed = jnp.bitwise_or(jnp.left_shift(xi[:, 1::2], 16), xi[:, 0::2])
    return lax.bitcast_convert_type(packed, jnp.int32)


_PERM = np.zeros((D,), np.int32)
for _v in range(D // 32):
    for _k in range(16):
        _PERM[32 * _v + _k] = 32 * _v + 2 * _k
        _PERM[32 * _v + 16 + _k] = 32 * _v + 2 * _k + 1


def kernel(metadata_entry, brand_entry, category_entry, price_entry,
           user_product_match_entry, program_types_input,
           W_meta, W_brand, W_cat, W_upm, W_dense, b_dense):
    midx = metadata_entry.astype(jnp.int32).reshape(
        NW * N_CHUNKS, IDX_ROWS, 128)
    bidx = brand_entry.astype(jnp.int32).reshape(N)
    cidx = category_entry.astype(jnp.int32).reshape(N)
    uidx = user_product_match_entry.astype(jnp.int32).reshape(N)
    pidx = program_types_input.astype(jnp.int32).reshape(N)
    price = price_entry.astype(jnp.float32).reshape(N)
    wdsub = W_dense[D:D + 1 + 32]
    z1, z2 = _sc_gather_sum(_pack_bf16(W_meta), _pack_bf16(W_brand),
                            _pack_bf16(W_cat), W_upm, wdsub,
                            midx, bidx, cidx, uidx, pidx, price)
    w2 = jnp.concatenate([W_dense[:D][_PERM], W_dense[D + 1 + 32:]], axis=0)
    return _tc_dense(z1, z2, w2, b_dense.reshape(1, D))


# speedup vs baseline: 3.0717x; 3.0717x over previous
import functools

import numpy as np

import jax
import jax.numpy as jnp
from jax import lax
from jax.experimental import pallas as pl
from jax.experimental.pallas import tpu as pltpu
from jax.experimental.pallas import tpu_sc as plsc

B, L, T, D = 1024, 50, 20, 64
N = B * L
NC, NS = 2, 16
NW = NC * NS
POS_PER_W = N // NW
CHUNK = 32
N_CHUNKS = POS_PER_W // CHUNK
IDX_ROWS = (CHUNK * T) // 128
UPM_V = 102
DP = D // 2


def _bf16_halves(w):
    ev = plsc.bitcast(jnp.left_shift(w, 16), jnp.float32)
    od = plsc.bitcast(jnp.bitwise_and(w, jnp.int32(-65536)), jnp.float32)
    return ev, od


def _sc_body(wmeta, wbrand, wcat, wupm, wdsub, midx, bidx, cidx,
             uidx, pidx, price, z1_out, z2_out,
             midx_v, bidx_all, cidx_all, uidx_all, pidx_all, price_all,
             wupm_v, wdsub_v, gbuf, bbuf, cbuf, z1buf, z2buf,
             sem_g, sem_i, sem_o):
    wid = lax.axis_index("s") * NC + lax.axis_index("c")
    base = wid * POS_PER_W
    chunk_base = wid * N_CHUNKS

    def fire_chunk(g_rel, p):
        for j in range(IDX_ROWS):
            pltpu.async_copy(
                wmeta.at[midx_v.at[p, j]],
                gbuf.at[p].at[pl.ds(j * 128, 128)], sem_g)
        loc = pl.ds(g_rel * CHUNK, CHUNK)
        pltpu.async_copy(wbrand.at[bidx_all.at[loc]], bbuf.at[p], sem_g)
        pltpu.async_copy(wcat.at[cidx_all.at[loc]], cbuf.at[p], sem_g)

    def wait_chunk(p):
        for j in range(IDX_ROWS):
            pltpu.make_async_copy(
                wmeta.at[midx_v.at[p, j]],
                gbuf.at[p].at[pl.ds(j * 128, 128)], sem_g).wait()
        pltpu.make_async_copy(wbrand.at[bidx_all.at[pl.ds(0, CHUNK)]],
                              bbuf.at[p], sem_g).wait()
        pltpu.make_async_copy(wcat.at[cidx_all.at[pl.ds(0, CHUNK)]],
                              cbuf.at[p], sem_g).wait()

    pltpu.sync_copy(bidx.at[pl.ds(base, POS_PER_W)], bidx_all)
    pltpu.sync_copy(cidx.at[pl.ds(base, POS_PER_W)], cidx_all)
    pltpu.sync_copy(uidx.at[pl.ds(base, POS_PER_W)],
                    uidx_all.at[pl.ds(0, POS_PER_W)])
    pltpu.sync_copy(pidx.at[pl.ds(base, POS_PER_W)],
                    pidx_all.at[pl.ds(0, POS_PER_W)])
    pltpu.sync_copy(price.at[pl.ds(base, POS_PER_W)],
                    price_all.at[pl.ds(0, POS_PER_W)])
    pltpu.sync_copy(wupm, wupm_v)
    pltpu.sync_copy(wdsub, wdsub_v)

    zeros16 = jnp.zeros((16,), jnp.float32)

    def zero_body(i, carry):
        for pp in range(2):
            for u in range(D // 16):
                z2buf[pp, i, pl.ds(D + u * 16, 16)] = zeros16
        return carry

    lax.fori_loop(0, CHUNK, zero_body, 0)

    pltpu.sync_copy(midx.at[chunk_base], midx_v.at[0])
    fire_chunk(0, 0)
    pltpu.async_copy(midx.at[chunk_base + 1], midx_v.at[1], sem_i)

    r_vecs = [wdsub_v[0, pl.ds(u * 16, 16)] for u in range(D // 16)]

    def outer_body(gg, carry):
        for b in range(2):
            g = gg * 2 + b
            p, np_ = b, 1 - b
            wait_chunk(p)

            @pl.when(g < N_CHUNKS - 1)
            def _():
                pltpu.make_async_copy(
                    midx.at[chunk_base + 1], midx_v.at[np_], sem_i).wait()
                fire_chunk(g + 1, np_)

            @pl.when(g < N_CHUNKS - 2)
            def _():
                pltpu.async_copy(
                    midx.at[chunk_base + g + 2], midx_v.at[p], sem_i)

            @pl.when(g >= 2)
            def _():
                pltpu.make_async_copy(
                    z1buf.at[p], z1_out.at[pl.ds(base, CHUNK)], sem_o).wait()
                pltpu.make_async_copy(
                    z2buf.at[p], z2_out.at[pl.ds(base, CHUNK)], sem_o).wait()

            def pos_body(i, carry2):
                base_row = i * T
                loc = g * CHUNK + i
                pu = uidx_all[pl.ds(loc, 16)][0]
                pg = pidx_all[pl.ds(loc, 16)][0]
                pr = price_all[pl.ds(loc, 16)][0]
                for v in range(DP // 16):
                    col = pl.ds(v * 16, 16)
                    ae, ao = _bf16_halves(gbuf[p, base_row, col])
                    for t in range(1, T):
                        he, ho = _bf16_halves(gbuf[p, base_row + t, col])
                        ae = ae + he
                        ao = ao + ho
                    be, bo = _bf16_halves(bbuf[p, i, col])
                    ce, co = _bf16_halves(cbuf[p, i, col])
                    z1buf[p, i, pl.ds(v * 32, 16)] = (
                        ae * (1.0 / T) + be + ce)
                    z1buf[p, i, pl.ds(v * 32 + 16, 16)] = (
                        ao * (1.0 / T) + bo + co)
                for u in range(D // 16):
                    col = pl.ds(u * 16, 16)
                    z1buf[p, i, pl.ds(D + u * 16, 16)] = wupm_v[pu, col]
                    z2buf[p, i, col] = wdsub_v[1 + pg, col] + pr * r_vecs[u]
                return carry2

            lax.fori_loop(0, CHUNK, pos_body, 0)
            pltpu.async_copy(
                z1buf.at[p], z1_out.at[pl.ds(base + g * CHUNK, CHUNK)], sem_o)
            pltpu.async_copy(
                z2buf.at[p], z2_out.at[pl.ds(base + g * CHUNK, CHUNK)], sem_o)
        return carry

    lax.fori_loop(0, N_CHUNKS // 2, outer_body, 0)
    for _ in range(2):
        pltpu.make_async_copy(
            z1buf.at[0], z1_out.at[pl.ds(base, CHUNK)], sem_o).wait()
        pltpu.make_async_copy(
            z2buf.at[0], z2_out.at[pl.ds(base, CHUNK)], sem_o).wait()


@jax.jit
def _sc_gather_sum(wmeta, wbrand, wcat, wupm, wdsub, midx, bidx, cidx,
                   uidx, pidx, price):
    mesh = plsc.VectorSubcoreMesh(core_axis_name="c", subcore_axis_name="s")
    return pl.kernel(
        _sc_body,
        out_type=(jax.ShapeDtypeStruct((N, 2 * D), jnp.float32),
                  jax.ShapeDtypeStruct((N, 2 * D), jnp.float32)),
        mesh=mesh,
        compiler_params=pltpu.CompilerParams(use_tc_tiling_on_sc=False,
                                             needs_layout_passes=False),
        scratch_types=[
            pltpu.VMEM((2, IDX_ROWS, 128), jnp.int32),
            pltpu.VMEM((POS_PER_W,), jnp.int32),
            pltpu.VMEM((POS_PER_W,), jnp.int32),
            pltpu.VMEM((POS_PER_W + 16,), jnp.int32),
            pltpu.VMEM((POS_PER_W + 16,), jnp.int32),
            pltpu.VMEM((POS_PER_W + 16,), jnp.float32),
            pltpu.VMEM((UPM_V, D), jnp.float32),
            pltpu.VMEM((33, D), jnp.float32),
            pltpu.VMEM((2, CHUNK * T, DP), jnp.int32),
            pltpu.VMEM((2, CHUNK, DP), jnp.int32),
            pltpu.VMEM((2, CHUNK, DP), jnp.int32),
            pltpu.VMEM((2, CHUNK, 2 * D), jnp.float32),
            pltpu.VMEM((2, CHUNK, 2 * D), jnp.float32),
            pltpu.SemaphoreType.DMA,
            pltpu.SemaphoreType.DMA,
            pltpu.SemaphoreType.DMA,
        ],
    )(wmeta, wbrand, wcat, wupm, wdsub, midx, bidx, cidx, uidx, pidx, price)


RB = 64
TC_R = RB * L


def _tc_body(z1_ref, z2_ref, w2_ref, b_ref, o_ref):
    acc = jnp.dot(z1_ref[:], w2_ref[:], preferred_element_type=jnp.float32)
    acc += z2_ref[:][:, :D]
    acc += b_ref[:]
    o_ref[:] = acc.reshape(RB, L, D)


@jax.jit
def _tc_dense(z1, z2, w2, b):
    return pl.pallas_call(
        _tc_body,
        grid=(B // RB,),
        in_specs=[
            pl.BlockSpec((TC_R, 2 * D), lambda i: (i, 0)),
            pl.BlockSpec((TC_R, 2 * D), lambda i: (i, 0)),
            pl.BlockSpec((2 * D, D), lambda i: (0, 0)),
            pl.BlockSpec((1, D), lambda i: (0, 0)),
        ],
        out_specs=pl.BlockSpec((RB, L, D), lambda i: (i, 0, 0)),
        out_shape=jax.ShapeDtypeStruct((B, L, D), jnp.float32),
    )(z1, z2, w2, b)


def _pack_bf16(w):
    v = w.shape[0]
    return lax.bitcast_convert_type(
        w.astype(jnp.bfloat16).reshape(v, DP, 2), jnp.int32)


_PERM = np.zeros((D,), np.int32)
for _v in range(D // 32):
    for _k in range(16):
        _PERM[32 * _v + _k] = 32 * _v + 2 * _k
        _PERM[32 * _v + 16 + _k] = 32 * _v + 2 * _k + 1


def kernel(metadata_entry, brand_entry, category_entry, price_entry,
           user_product_match_entry, program_types_input,
           W_meta, W_brand, W_cat, W_upm, W_dense, b_dense):
    midx = metadata_entry.astype(jnp.int32).reshape(
        NW * N_CHUNKS, IDX_ROWS, 128)
    bidx = brand_entry.astype(jnp.int32).reshape(N)
    cidx = category_entry.astype(jnp.int32).reshape(N)
    uidx = user_product_match_entry.astype(jnp.int32).reshape(N)
    pidx = program_types_input.astype(jnp.int32).reshape(N)
    price = price_entry.astype(jnp.float32).reshape(N)
    wdsub = W_dense[D:D + 1 + 32]
    z1, z2 = _sc_gather_sum(_pack_bf16(W_meta), _pack_bf16(W_brand),
                            _pack_bf16(W_cat), W_upm, wdsub,
                            midx, bidx, cidx, uidx, pidx, price)
    w2 = jnp.concatenate([W_dense[:D][_PERM], W_dense[D + 1 + 32:]], axis=0)
    return _tc_dense(z1, z2, w2, b_dense.reshape(1, D))


# speedup vs baseline: 5.0713x; 1.6510x over previous
import functools

import numpy as np

import jax
import jax.numpy as jnp
from jax import lax
from jax.experimental import pallas as pl
from jax.experimental.pallas import tpu as pltpu
from jax.experimental.pallas import tpu_sc as plsc

B, L, T, D = 1024, 50, 20, 64
N = B * L
NC, NS = 2, 16
NW = NC * NS
POS_PER_W = N // NW
CHUNK = 32
N_CHUNKS = POS_PER_W // CHUNK
IDX_ROWS = (CHUNK * T) // 128
UPM_V = 102
DP = D // 2


def _bf16_halves(w):
    lo = plsc.bitcast(jnp.left_shift(w, 16), jnp.float32)
    hi = plsc.bitcast(jnp.bitwise_and(w, jnp.int32(-65536)), jnp.float32)
    return lo, hi


def _sc_body(wmeta, wbrand, wcat, wupm, wdsub, midx, bidx, cidx,
             uidx, pidx, price, z1_out, z2_out,
             midx_v, bidx_all, cidx_all, uidx_all, pidx_all, price_all,
             wupm_v, wdsub_v, gbuf, bbuf, cbuf, z1buf, z2buf,
             sem_g, sem_i, sem_o):
    wid = lax.axis_index("s") * NC + lax.axis_index("c")
    base = wid * POS_PER_W
    chunk_base = wid * N_CHUNKS

    def fire_chunk(g_rel, p):
        for j in range(IDX_ROWS):
            pltpu.async_copy(
                wmeta.at[midx_v.at[p, j]],
                gbuf.at[p].at[pl.ds(j * 128, 128)], sem_g)
        loc = pl.ds(g_rel * CHUNK, CHUNK)
        pltpu.async_copy(wbrand.at[bidx_all.at[loc]], bbuf.at[p], sem_g)
        pltpu.async_copy(wcat.at[cidx_all.at[loc]], cbuf.at[p], sem_g)

    def wait_chunk(p):
        for j in range(IDX_ROWS):
            pltpu.make_async_copy(
                wmeta.at[midx_v.at[p, j]],
                gbuf.at[p].at[pl.ds(j * 128, 128)], sem_g).wait()
        pltpu.make_async_copy(wbrand.at[bidx_all.at[pl.ds(0, CHUNK)]],
                              bbuf.at[p], sem_g).wait()
        pltpu.make_async_copy(wcat.at[cidx_all.at[pl.ds(0, CHUNK)]],
                              cbuf.at[p], sem_g).wait()

    pltpu.sync_copy(bidx.at[pl.ds(base, POS_PER_W)], bidx_all)
    pltpu.sync_copy(cidx.at[pl.ds(base, POS_PER_W)], cidx_all)
    pltpu.sync_copy(uidx.at[pl.ds(base, POS_PER_W)],
                    uidx_all.at[pl.ds(0, POS_PER_W)])
    pltpu.sync_copy(pidx.at[pl.ds(base, POS_PER_W)],
                    pidx_all.at[pl.ds(0, POS_PER_W)])
    pltpu.sync_copy(price.at[pl.ds(base, POS_PER_W)],
                    price_all.at[pl.ds(0, POS_PER_W)])
    pltpu.sync_copy(wupm, wupm_v)
    pltpu.sync_copy(wdsub, wdsub_v)

    zeros16 = jnp.zeros((16,), jnp.float32)

    def zero_body(i, carry):
        for pp in range(2):
            for u in range(D // 16):
                z2buf[pp, i, pl.ds(D + u * 16, 16)] = zeros16
        return carry

    lax.fori_loop(0, CHUNK, zero_body, 0)

    pltpu.sync_copy(midx.at[chunk_base], midx_v.at[0])
    fire_chunk(0, 0)
    pltpu.async_copy(midx.at[chunk_base + 1], midx_v.at[1], sem_i)

    r_vecs = [wdsub_v[0, pl.ds(u * 16, 16)] for u in range(D // 16)]

    def outer_body(gg, carry):
        for b in range(2):
            g = gg * 2 + b
            p, np_ = b, 1 - b
            wait_chunk(p)

            @pl.when(g < N_CHUNKS - 1)
            def _():
                pltpu.make_async_copy(
                    midx.at[chunk_base + 1], midx_v.at[np_], sem_i).wait()
                fire_chunk(g + 1, np_)

            @pl.when(g < N_CHUNKS - 2)
            def _():
                pltpu.async_copy(
                    midx.at[chunk_base + g + 2], midx_v.at[p], sem_i)

            @pl.when(g >= 2)
            def _():
                pltpu.make_async_copy(
                    z1buf.at[p], z1_out.at[pl.ds(base, CHUNK)], sem_o).wait()
                pltpu.make_async_copy(
                    z2buf.at[p], z2_out.at[pl.ds(base, CHUNK)], sem_o).wait()

            def pos_body(i, carry2):
                base_row = i * T
                loc = g * CHUNK + i
                pu = uidx_all[pl.ds(loc, 16)][0]
                pg = pidx_all[pl.ds(loc, 16)][0]
                pr = price_all[pl.ds(loc, 16)][0]
                for v in range(DP // 16):
                    col = pl.ds(v * 16, 16)
                    ae, ao = _bf16_halves(gbuf[p, base_row, col])
                    for t in range(1, T):
                        he, ho = _bf16_halves(gbuf[p, base_row + t, col])
                        ae = ae + he
                        ao = ao + ho
                    be, bo = _bf16_halves(bbuf[p, i, col])
                    ce, co = _bf16_halves(cbuf[p, i, col])
                    z1buf[p, i, pl.ds(v * 16, 16)] = (
                        ae * (1.0 / T) + be + ce)
                    z1buf[p, i, pl.ds(32 + v * 16, 16)] = (
                        ao * (1.0 / T) + bo + co)
                for u in range(D // 16):
                    col = pl.ds(u * 16, 16)
                    z1buf[p, i, pl.ds(D + u * 16, 16)] = wupm_v[pu, col]
                    z2buf[p, i, col] = wdsub_v[1 + pg, col] + pr * r_vecs[u]
                return carry2

            lax.fori_loop(0, CHUNK, pos_body, 0)
            pltpu.async_copy(
                z1buf.at[p], z1_out.at[pl.ds(base + g * CHUNK, CHUNK)], sem_o)
            pltpu.async_copy(
                z2buf.at[p], z2_out.at[pl.ds(base + g * CHUNK, CHUNK)], sem_o)
        return carry

    lax.fori_loop(0, N_CHUNKS // 2, outer_body, 0)
    for _ in range(2):
        pltpu.make_async_copy(
            z1buf.at[0], z1_out.at[pl.ds(base, CHUNK)], sem_o).wait()
        pltpu.make_async_copy(
            z2buf.at[0], z2_out.at[pl.ds(base, CHUNK)], sem_o).wait()


@jax.jit
def _sc_gather_sum(wmeta, wbrand, wcat, wupm, wdsub, midx, bidx, cidx,
                   uidx, pidx, price):
    mesh = plsc.VectorSubcoreMesh(core_axis_name="c", subcore_axis_name="s")
    return pl.kernel(
        _sc_body,
        out_type=(jax.ShapeDtypeStruct((N, 2 * D), jnp.float32),
                  jax.ShapeDtypeStruct((N, 2 * D), jnp.float32)),
        mesh=mesh,
        compiler_params=pltpu.CompilerParams(use_tc_tiling_on_sc=False,
                                             needs_layout_passes=False),
        scratch_types=[
            pltpu.VMEM((2, IDX_ROWS, 128), jnp.int32),
            pltpu.VMEM((POS_PER_W,), jnp.int32),
            pltpu.VMEM((POS_PER_W,), jnp.int32),
            pltpu.VMEM((POS_PER_W + 16,), jnp.int32),
            pltpu.VMEM((POS_PER_W + 16,), jnp.int32),
            pltpu.VMEM((POS_PER_W + 16,), jnp.float32),
            pltpu.VMEM((UPM_V, D), jnp.float32),
            pltpu.VMEM((33, D), jnp.float32),
            pltpu.VMEM((2, CHUNK * T, DP), jnp.int32),
            pltpu.VMEM((2, CHUNK, DP), jnp.int32),
            pltpu.VMEM((2, CHUNK, DP), jnp.int32),
            pltpu.VMEM((2, CHUNK, 2 * D), jnp.float32),
            pltpu.VMEM((2, CHUNK, 2 * D), jnp.float32),
            pltpu.SemaphoreType.DMA,
            pltpu.SemaphoreType.DMA,
            pltpu.SemaphoreType.DMA,
        ],
    )(wmeta, wbrand, wcat, wupm, wdsub, midx, bidx, cidx, uidx, pidx, price)


RB = 64
TC_R = RB * L


def _tc_body(z1_ref, z2_ref, w2_ref, b_ref, o_ref):
    acc = jnp.dot(z1_ref[:], w2_ref[:], preferred_element_type=jnp.float32)
    acc += z2_ref[:][:, :D]
    acc += b_ref[:]
    o_ref[:] = acc.reshape(RB, L, D)


@jax.jit
def _tc_dense(z1, z2, w2, b):
    return pl.pallas_call(
        _tc_body,
        grid=(B // RB,),
        in_specs=[
            pl.BlockSpec((TC_R, 2 * D), lambda i: (i, 0)),
            pl.BlockSpec((TC_R, 2 * D), lambda i: (i, 0)),
            pl.BlockSpec((2 * D, D), lambda i: (0, 0)),
            pl.BlockSpec((1, D), lambda i: (0, 0)),
        ],
        out_specs=pl.BlockSpec((RB, L, D), lambda i: (i, 0, 0)),
        out_shape=jax.ShapeDtypeStruct((B, L, D), jnp.float32),
    )(z1, z2, w2, b)


def _pack_bf16(w):
    xb = w.astype(jnp.bfloat16)
    lo = lax.bitcast_convert_type(xb[:, :DP], jnp.uint16).astype(jnp.uint32)
    hi = lax.bitcast_convert_type(xb[:, DP:], jnp.uint16).astype(jnp.uint32)
    return lax.bitcast_convert_type(
        jnp.bitwise_or(jnp.left_shift(hi, 16), lo), jnp.int32)


def kernel(metadata_entry, brand_entry, category_entry, price_entry,
           user_product_match_entry, program_types_input,
           W_meta, W_brand, W_cat, W_upm, W_dense, b_dense):
    midx = metadata_entry.astype(jnp.int32).reshape(
        NW * N_CHUNKS, IDX_ROWS, 128)
    bidx = brand_entry.astype(jnp.int32).reshape(N)
    cidx = category_entry.astype(jnp.int32).reshape(N)
    uidx = user_product_match_entry.astype(jnp.int32).reshape(N)
    pidx = program_types_input.astype(jnp.int32).reshape(N)
    price = price_entry.astype(jnp.float32).reshape(N)
    wdsub = W_dense[D:D + 1 + 32]
    z1, z2 = _sc_gather_sum(_pack_bf16(W_meta), _pack_bf16(W_brand),
                            _pack_bf16(W_cat), W_upm, wdsub,
                            midx, bidx, cidx, uidx, pidx, price)
    w2 = jnp.concatenate([W_dense[:D], W_dense[D + 1 + 32:]], axis=0)
    return _tc_dense(z1, z2, w2, b_dense.reshape(1, D))
